# baseline (device time: 899447 ns/iter reference)
import jax
import jax.numpy as jnp
from jax import lax
from jax.experimental import pallas as pl
from jax.experimental.pallas import tpu as pltpu

jax.config.update("jax_compilation_cache_dir", "/tmp/jax_comp_cache")
jax.config.update("jax_persistent_cache_min_compile_time_secs", 1.0)

T, D, V = 2048, 4096, 16384
RB = T // 4
CB = V // 2
NC = 16
WC = CB // NC
NCP = 4
PW = CB // NCP
NQ = 4096


def kernel(x, W):
    xi = lax.axis_index("x")
    zi = lax.axis_index("z")
    x_rows = lax.dynamic_slice_in_dim(x, (2 * xi + zi) * RB, RB, axis=0)

    def body(x_ref, w_ref, out_ref, wbuf, piece, stage, st_mine, st_peer,
             st_all, wsems, lsems, zsend, zrecv, ysend, yrecv, xsend, xrecv,
             fsend, frecv, ssend, srecv, stage_sem):
        xi = lax.axis_index("x")
        yi = lax.axis_index("y")
        zi = lax.axis_index("z")
        row0 = (2 * xi + zi) * RB
        rowz = (2 * xi + (1 - zi)) * RB
        rowx0 = (2 * (1 - xi) + zi) * RB
        xrow0 = xi * (2 * RB)
        col0 = yi * CB
        colo = (1 - yi) * CB
        z_peer = (xi, yi, 1 - zi)
        y_peer = (xi, 1 - yi, zi)
        x_peer = (1 - xi, yi, zi)

        def rdma(src, dst, ssem, rsem, peer):
            return pltpu.make_async_remote_copy(
                src_ref=src, dst_ref=dst, send_sem=ssem, recv_sem=rsem,
                device_id=peer, device_id_type=pl.DeviceIdType.MESH)

        def wload(c):
            return pltpu.make_async_copy(
                w_ref.at[:, pl.ds(c * WC, WC)], wbuf.at[c % 2],
                wsems.at[c % 2])

        wload(0).start()

        barrier = pltpu.get_barrier_semaphore()
        for nbr in [x_peer, y_peer, z_peer]:
            pl.semaphore_signal(barrier, inc=1, device_id=nbr,
                                device_id_type=pl.DeviceIdType.MESH)
        pl.semaphore_wait(barrier, 3)

        x_val = x_ref[:, :].astype(jnp.bfloat16)
        z_h = [None] * NCP
        y_h = [None] * NCP
        x_h = [None] * NCP
        l_h = [None] * NCP
        m_run = None
        s_run = None
        cpp = NC // NCP

        for c in range(NC):
            p = c // cpp
            pltpu.make_async_copy(
                w_ref.at[:, pl.ds(c * WC, WC)], wbuf.at[c % 2],
                wsems.at[c % 2]).wait()
            if c + 1 < NC:
                wload(c + 1).start()
            if c % cpp == 0 and p >= 2:
                z_h[p - 2].wait_send()
                y_h[p - 2].wait_send()
                x_h[p - 2].wait_send()
                l_h[p - 2].wait()
            l_c = jnp.dot(x_val, wbuf[c % 2, :, :].astype(jnp.bfloat16),
                          preferred_element_type=jnp.float32)
            piece[p % 2, :, pl.ds((c % cpp) * WC, WC)] = l_c
            mc = jnp.max(l_c, axis=1, keepdims=True)
            if c == 0:
                m_run = mc
                s_run = jnp.sum(jnp.exp(l_c - mc), axis=1, keepdims=True)
            else:
                m_new = jnp.maximum(m_run, mc)
                s_run = (s_run * jnp.exp(m_run - m_new)
                         + jnp.sum(jnp.exp(l_c - m_new), axis=1, keepdims=True))
                m_run = m_new
            if c % cpp == cpp - 1:
                dst = out_ref.at[pl.ds(row0, RB), pl.ds(col0 + p * PW, PW)]
                src = piece.at[p % 2]
                l_h[p] = pltpu.make_async_copy(src, dst, lsems.at[p])
                l_h[p].start()
                z_h[p] = rdma(src, dst, zsend.at[p], zrecv.at[p], z_peer)
                y_h[p] = rdma(src, dst, ysend.at[p], yrecv.at[p], y_peer)
                x_h[p] = rdma(src, dst, xsend.at[p], xrecv.at[p], x_peer)
                z_h[p].start()
                y_h[p].start()
                x_h[p].start()

        for p in range(NCP - 2, NCP):
            z_h[p].wait_send()
            y_h[p].wait_send()
            x_h[p].wait_send()
            l_h[p].wait()

        st_mine[:, 0:128] = jnp.broadcast_to(m_run, (RB, 128))
        st_mine[:, 128:256] = jnp.broadcast_to(s_run, (RB, 128))
        sy = rdma(st_mine, st_peer, ssend.at[0], srecv.at[0], y_peer)
        sy.start()
        sy.wait()
        m_o = st_peer[:, 0:1]
        s_o = st_peer[:, 128:129]
        m_g = jnp.maximum(m_run, m_o)
        s_g = s_run * jnp.exp(m_run - m_g) + s_o * jnp.exp(m_o - m_g)
        st_all[pl.ds(row0, RB), 0:128] = jnp.broadcast_to(m_g, (RB, 128))
        st_all[pl.ds(row0, RB), 128:256] = jnp.broadcast_to(1.0 / s_g, (RB, 128))
        sz = rdma(st_all.at[pl.ds(row0, RB)], st_all.at[pl.ds(row0, RB)],
                  ssend.at[1], srecv.at[1], z_peer)
        sz.start()
        sz.wait()
        sx = rdma(st_all.at[pl.ds(xrow0, 2 * RB)],
                  st_all.at[pl.ds(xrow0, 2 * RB)],
                  ssend.at[2], srecv.at[2], x_peer)
        sx.start()
        sx.wait()

        def norm_region(rs, chalf):
            for q in range(CB // NQ):
                cs = chalf + q * NQ
                ld = pltpu.make_async_copy(
                    out_ref.at[pl.ds(rs, RB), pl.ds(cs, NQ)], stage, stage_sem)
                ld.start()
                ld.wait()
                mm = st_all[pl.ds(rs, RB), 0:1]
                iv = st_all[pl.ds(rs, RB), 128:129]
                stage[:, :] = jnp.exp(stage[:, :] - mm) * iv
                st = pltpu.make_async_copy(
                    stage, out_ref.at[pl.ds(rs, RB), pl.ds(cs, NQ)], stage_sem)
                st.start()
                st.wait()

        norm_region(row0, col0)

        for p in range(NCP):
            z_h[p].wait_recv()
        norm_region(rowz, col0)
        y_b = rdma(out_ref.at[pl.ds(rowz, RB), pl.ds(col0, CB)],
                   out_ref.at[pl.ds(rowz, RB), pl.ds(col0, CB)],
                   fsend.at[0], frecv.at[0], y_peer)
        x_b = rdma(out_ref.at[pl.ds(rowz, RB), pl.ds(col0, CB)],
                   out_ref.at[pl.ds(rowz, RB), pl.ds(col0, CB)],
                   fsend.at[1], frecv.at[1], x_peer)
        y_b.start()
        x_b.start()

        for p in range(NCP):
            y_h[p].wait_recv()
        norm_region(row0, colo)
        x_ca = rdma(out_ref.at[pl.ds(row0, RB), pl.ds(colo, CB)],
                    out_ref.at[pl.ds(row0, RB), pl.ds(colo, CB)],
                    fsend.at[2], frecv.at[2], x_peer)
        x_ca.start()

        for p in range(NCP):
            x_h[p].wait_recv()
        norm_region(rowx0, col0)

        y_b.wait_recv()
        x_cb = rdma(out_ref.at[pl.ds(rowz, RB), pl.ds(colo, CB)],
                    out_ref.at[pl.ds(rowz, RB), pl.ds(colo, CB)],
                    fsend.at[3], frecv.at[3], x_peer)
        x_cb.start()

        x_b.wait_recv()
        x_ca.wait_recv()
        x_cb.wait_recv()
        y_b.wait_send()
        x_b.wait_send()
        x_ca.wait_send()
        x_cb.wait_send()

    return pl.pallas_call(
        body,
        out_shape=jax.ShapeDtypeStruct((T, V), jnp.float32),
        in_specs=[pl.BlockSpec(memory_space=pltpu.MemorySpace.VMEM),
                  pl.BlockSpec(memory_space=pl.ANY)],
        out_specs=pl.BlockSpec(memory_space=pltpu.MemorySpace.HBM),
        scratch_shapes=[
            pltpu.VMEM((2, D, WC), jnp.float32),
            pltpu.VMEM((2, RB, PW), jnp.float32),
            pltpu.VMEM((RB, NQ), jnp.float32),
            pltpu.VMEM((RB, 256), jnp.float32),
            pltpu.VMEM((RB, 256), jnp.float32),
            pltpu.VMEM((T, 256), jnp.float32),
            pltpu.SemaphoreType.DMA((2,)),
            pltpu.SemaphoreType.DMA((NCP,)),
            pltpu.SemaphoreType.DMA((NCP,)),
            pltpu.SemaphoreType.DMA((NCP,)),
            pltpu.SemaphoreType.DMA((NCP,)),
            pltpu.SemaphoreType.DMA((NCP,)),
            pltpu.SemaphoreType.DMA((NCP,)),
            pltpu.SemaphoreType.DMA((NCP,)),
            pltpu.SemaphoreType.DMA((4,)),
            pltpu.SemaphoreType.DMA((4,)),
            pltpu.SemaphoreType.DMA((3,)),
            pltpu.SemaphoreType.DMA((3,)),
            pltpu.SemaphoreType.DMA,
        ],
        compiler_params=pltpu.CompilerParams(
            collective_id=0, vmem_limit_bytes=60 * 1024 * 1024),
    )(x_rows, W)
